# baseline (speedup 1.0000x reference)
import functools

import jax
import jax.numpy as jnp
from jax import lax
from jax.experimental import pallas as pl
from jax.experimental.pallas import tpu as pltpu
from jax.experimental.pallas import tpu_sc as plsc


def _make_sc_gather(V, D, NW, n_chunks, C):
    mesh = plsc.VectorSubcoreMesh(core_axis_name="c", subcore_axis_name="s")
    info = plsc.get_sparse_core_info()
    NC = info.num_cores

    @functools.partial(
        pl.kernel,
        mesh=mesh,
        compiler_params=pltpu.CompilerParams(use_tc_tiling_on_sc=False),
        out_type=jax.ShapeDtypeStruct((NW, n_chunks, C, D), jnp.float32),
        scratch_types=[
            pltpu.VMEM((n_chunks, C), jnp.int32),
            pltpu.VMEM((C, D), jnp.float32),
            pltpu.SemaphoreType.DMA,
        ],
    )
    def gather(idx_hbm, table_hbm, out_hbm, idx_v, rows_v, gsem):
        wid = lax.axis_index("s") * NC + lax.axis_index("c")
        pltpu.sync_copy(idx_hbm.at[wid], idx_v)

        def body(j, carry):
            pltpu.async_copy(table_hbm.at[idx_v.at[j]], rows_v, gsem).wait()
            pltpu.sync_copy(rows_v, out_hbm.at[wid, j])
            return carry

        lax.fori_loop(0, n_chunks, body, 0)

    return gather


def kernel(input, table):
    B, S = input.shape
    V, D = table.shape
    N = B * S
    NW = 32
    C = 128
    per_w = N // NW
    n_chunks = per_w // C

    idx = input.reshape(NW, n_chunks, C).astype(jnp.int32)
    out = _make_sc_gather(V, D, NW, n_chunks, C)(idx, table)
    return out.reshape(B, S, D)


# speedup vs baseline: 1.0459x; 1.0459x over previous
import functools

import jax
import jax.numpy as jnp
from jax import lax
from jax.experimental import pallas as pl
from jax.experimental.pallas import tpu as pltpu
from jax.experimental.pallas import tpu_sc as plsc


def _make_sc_gather(V, D, NW, n_chunks, C, NB):
    mesh = plsc.VectorSubcoreMesh(core_axis_name="c", subcore_axis_name="s")
    info = plsc.get_sparse_core_info()
    NC = info.num_cores
    n_outer = n_chunks // NB

    @functools.partial(
        pl.kernel,
        mesh=mesh,
        compiler_params=pltpu.CompilerParams(use_tc_tiling_on_sc=False),
        out_type=jax.ShapeDtypeStruct((NW, n_chunks, C, D), jnp.float32),
        scratch_types=[
            pltpu.VMEM((n_chunks, C), jnp.int32),
            pltpu.VMEM((NB, C, D), jnp.float32),
            pltpu.SemaphoreType.DMA((NB,)),
            pltpu.SemaphoreType.DMA((NB,)),
        ],
    )
    def gather(idx_hbm, table_hbm, out_hbm, idx_v, rows_v, gsem, ssem):
        wid = lax.axis_index("s") * NC + lax.axis_index("c")
        pltpu.sync_copy(idx_hbm.at[wid], idx_v)

        def g_start(b, j):
            pltpu.async_copy(table_hbm.at[idx_v.at[j]], rows_v.at[b], gsem.at[b])

        def g_wait(b):
            pltpu.make_async_copy(
                table_hbm.at[idx_v.at[0]], rows_v.at[b], gsem.at[b]
            ).wait()

        def s_start(b, j):
            pltpu.async_copy(rows_v.at[b], out_hbm.at[wid, j], ssem.at[b])

        def s_wait(b):
            pltpu.make_async_copy(
                rows_v.at[b], out_hbm.at[wid, 0], ssem.at[b]
            ).wait()

        for b in range(NB):
            g_start(b, b)

        def outer(o, carry):
            for b in range(NB):
                g_wait(b)
                s_start(b, o * NB + b)
            for b in range(NB):
                s_wait(b)

                @pl.when(o < n_outer - 1)
                def _():
                    g_start(b, o * NB + b + NB)

            return carry

        lax.fori_loop(0, n_outer, outer, 0)

    return gather


def kernel(input, table):
    B, S = input.shape
    V, D = table.shape
    N = B * S
    NW = 32
    C = 128
    per_w = N // NW
    n_chunks = per_w // C
    NB = 5

    idx = input.reshape(NW, n_chunks, C).astype(jnp.int32)
    out = _make_sc_gather(V, D, NW, n_chunks, C, NB)(idx, table)
    return out.reshape(B, S, D)
